# baseline (speedup 1.0000x reference)
import jax
import jax.numpy as jnp
from jax.experimental import pallas as pl
from jax.experimental.pallas import tpu as pltpu


def _feature_kernel(x_ref, w1_ref, w2_ref, h2_ref):
    h = jnp.dot(x_ref[...], w1_ref[...], preferred_element_type=jnp.float32)
    h = jnp.maximum(h, 0.0)
    h2_ref[...] = jnp.dot(h, w2_ref[...], preferred_element_type=jnp.float32)


def _agg1_kernel(adj_ref, h2_ref, b1_ref, w3_ref, g_ref):
    t = jnp.dot(adj_ref[...], h2_ref[...], preferred_element_type=jnp.float32)
    t = t + b1_ref[...]
    g_ref[...] = jnp.dot(t, w3_ref[...], preferred_element_type=jnp.float32)


def _agg2_kernel(adj_ref, g_ref, b2_ref, out_ref):
    logits = jnp.dot(adj_ref[...], g_ref[...], preferred_element_type=jnp.float32)
    logits = logits + b2_ref[...]
    m = jnp.max(logits, axis=1, keepdims=True)
    lse = jnp.log(jnp.sum(jnp.exp(logits - m), axis=1, keepdims=True)) + m
    out_ref[...] = logits - lse


def _row_block(n: int, target: int = 400) -> int:
    best = 8
    for d in range(8, target + 1, 8):
        if n % d == 0:
            best = d
    return best


def kernel(x, adj, W1, W2, b1, W3, b2):
    n, _ = x.shape
    mid = W2.shape[1]
    ncls = W3.shape[1]
    b1r = b1.reshape(1, mid)
    b2r = b2.reshape(1, ncls)
    bm = _row_block(n)
    grid = (n // bm,)
    params = pltpu.CompilerParams(dimension_semantics=("parallel",))

    h2 = pl.pallas_call(
        _feature_kernel,
        out_shape=jax.ShapeDtypeStruct((n, mid), jnp.float32),
    )(x, W1, W2)

    g = pl.pallas_call(
        _agg1_kernel,
        grid=grid,
        in_specs=[
            pl.BlockSpec((bm, n), lambda i: (i, 0)),
            pl.BlockSpec((n, mid), lambda i: (0, 0)),
            pl.BlockSpec((1, mid), lambda i: (0, 0)),
            pl.BlockSpec((mid, ncls), lambda i: (0, 0)),
        ],
        out_specs=pl.BlockSpec((bm, ncls), lambda i: (i, 0)),
        out_shape=jax.ShapeDtypeStruct((n, ncls), jnp.float32),
        compiler_params=params,
    )(adj, h2, b1r, W3)

    out = pl.pallas_call(
        _agg2_kernel,
        grid=grid,
        in_specs=[
            pl.BlockSpec((bm, n), lambda i: (i, 0)),
            pl.BlockSpec((n, ncls), lambda i: (0, 0)),
            pl.BlockSpec((1, ncls), lambda i: (0, 0)),
        ],
        out_specs=pl.BlockSpec((bm, ncls), lambda i: (i, 0)),
        out_shape=jax.ShapeDtypeStruct((n, ncls), jnp.float32),
        compiler_params=params,
    )(adj, g, b2r)
    return out
